# Initial kernel scaffold; baseline (speedup 1.0000x reference)
import jax
import jax.numpy as jnp
from jax.experimental import pallas as pl


def kernel(tokens, memory, connections):
    raise NotImplementedError("write your pallas kernel here")


# speedup vs baseline: 367.5242x; 367.5242x over previous
import functools

import jax
import jax.numpy as jnp
from jax import lax
from jax.experimental import pallas as pl
from jax.experimental.pallas import tpu as pltpu
from jax.experimental.pallas import tpu_sc as plsc

S = 2048
IB = 512
M = 1024
NC = 2
NS = 16
LANES = 16
CW = IB // NC
RW = S // NS
MW = M // NS
ZR = 16


def _addr_body(tokens_ref, w_ref, out_ref):
    acc = jnp.dot(tokens_ref[...], w_ref[...],
                  preferred_element_type=jnp.float32)
    out_ref[...] = acc.astype(jnp.int32)


def _sc_body(tokens_hbm, aq_hbm, ak_hbm, mem_hbm, mq_hbm, mk_hbm, out_hbm,
             tokbuf, akbuf, aqbuf, membuf, mqbuf, mkbuf, P, U):
    c = lax.axis_index("c")
    s = lax.axis_index("s")
    t = c * NS + s
    c0 = t * LANES

    pltpu.sync_copy(tokens_hbm.at[:, pl.ds(c0, LANES)], tokbuf)
    pltpu.sync_copy(ak_hbm, akbuf)
    pltpu.sync_copy(aq_hbm, aqbuf)
    pltpu.sync_copy(mem_hbm, membuf)
    pltpu.sync_copy(mq_hbm, mqbuf)
    pltpu.sync_copy(mk_hbm, mkbuf)

    zrow = jnp.zeros((LANES,), jnp.float32)

    def _zero(r, carry):
        P[r, :] = zrow
        U[r, :] = zrow
        return carry

    lax.fori_loop(0, M, _zero, 0)
    U[M, :] = zrow

    def _scat(jj, carry):
        j0 = jj * LANES
        akv = akbuf[pl.ds(j0, LANES)]
        for l in range(LANES):
            akj = akv[l]
            P[akj, :] = P[akj, :] + tokbuf[j0 + l, :]
        return carry

    lax.fori_loop(0, S // LANES, _scat, 0)

    def _accum(mm, carry):
        m0 = mm * LANES
        memv = membuf[pl.ds(m0, LANES)]
        mqv = mqbuf[pl.ds(m0, LANES)]
        mkv = mkbuf[pl.ds(m0, LANES)]
        urowv = jnp.where(memv >= 0.5, mqv, M)
        for l in range(LANES):
            urow = urowv[l]
            U[urow, :] = U[urow, :] + P[mkv[l], :]
        return carry

    lax.fori_loop(0, M // LANES, _accum, 0)

    def _gath(ii, carry):
        i0 = ii * LANES
        aqv = aqbuf[pl.ds(i0, LANES)]
        for l in range(LANES):
            v = U[aqv[l], :]
            tokbuf[i0 + l, :] = jnp.bitwise_and(
                v.astype(jnp.int32), 1).astype(jnp.float32)
        return carry

    lax.fori_loop(0, S // LANES, _gath, 0)
    pltpu.sync_copy(tokbuf, out_hbm.at[:, pl.ds(c0, LANES)])


_sc_call = functools.partial(
    pl.kernel,
    mesh=plsc.VectorSubcoreMesh(core_axis_name="c", subcore_axis_name="s"),
    out_type=jax.ShapeDtypeStruct((S, IB), jnp.float32),
    compiler_params=pltpu.CompilerParams(use_tc_tiling_on_sc=False),
    scratch_types=[
        pltpu.VMEM((S, LANES), jnp.float32),
        pltpu.VMEM((S,), jnp.int32),
        pltpu.VMEM((S,), jnp.int32),
        pltpu.VMEM((M,), jnp.float32),
        pltpu.VMEM((M,), jnp.int32),
        pltpu.VMEM((M,), jnp.int32),
        pltpu.VMEM((M, LANES), jnp.float32),
        pltpu.VMEM((M + 1, LANES), jnp.float32),
    ],
)(_sc_body)


@jax.jit
def kernel(tokens, memory, connections):
    ib = tokens.shape[1]
    nb = connections.shape[0]
    powers = 2 ** jnp.arange(nb, dtype=jnp.int32)
    qm = connections < ib
    qpos = jnp.where(qm, connections, 0)
    kpos = jnp.where(qm, 0, connections - ib)
    wq_vals = jnp.where(qm, powers, 0).astype(jnp.float32)
    wk_vals = jnp.where(qm, 0, powers).astype(jnp.float32)
    wq = jnp.zeros((ib,), jnp.float32).at[qpos].add(wq_vals)
    wk = jnp.zeros((ib,), jnp.float32).at[kpos].add(wk_vals)
    W = jnp.zeros((ib, 128), jnp.float32).at[:, 0].set(wq).at[:, 1].set(wk)

    addr = pl.pallas_call(
        _addr_body,
        out_shape=jax.ShapeDtypeStruct((S, 128), jnp.int32),
    )(tokens, W)
    aq = addr[:, 0]
    ak = addr[:, 1]

    qmask = jnp.sum(powers * qm.astype(jnp.int32))
    marange = jnp.arange(M, dtype=jnp.int32)
    mq = jnp.bitwise_and(marange, qmask)
    mk = marange - mq

    return _sc_call(tokens, aq, ak, memory, mq, mk)


# speedup vs baseline: 381.3020x; 1.0375x over previous
import functools

import jax
import jax.numpy as jnp
from jax import lax
from jax.experimental import pallas as pl
from jax.experimental.pallas import tpu as pltpu
from jax.experimental.pallas import tpu_sc as plsc

S = 2048
IB = 512
M = 1024
NC = 2
NS = 16
LANES = 16
CW = IB // NC
RW = S // NS
MW = M // NS
ZR = 16


def _addr_body(tokens_ref, wq_ref, wk_ref, aq_ref, ak_ref):
    t = tokens_ref[...]
    aq_ref[...] = jnp.sum(t * wq_ref[...][None, :], axis=1).astype(jnp.int32)
    ak_ref[...] = jnp.sum(t * wk_ref[...][None, :], axis=1).astype(jnp.int32)


def _sc_body(tokens_hbm, aq_hbm, ak_hbm, mem_hbm, mq_hbm, mk_hbm, out_hbm,
             tokbuf, akbuf, aqbuf, membuf, mqbuf, mkbuf, P, U):
    c = lax.axis_index("c")
    s = lax.axis_index("s")
    t = c * NS + s
    c0 = t * LANES

    pltpu.sync_copy(tokens_hbm.at[:, pl.ds(c0, LANES)], tokbuf)
    pltpu.sync_copy(ak_hbm, akbuf)
    pltpu.sync_copy(aq_hbm, aqbuf)
    pltpu.sync_copy(mem_hbm, membuf)
    pltpu.sync_copy(mq_hbm, mqbuf)
    pltpu.sync_copy(mk_hbm, mkbuf)

    zrow = jnp.zeros((LANES,), jnp.float32)

    def _zero(r, carry):
        P[r, :] = zrow
        U[r, :] = zrow
        return carry

    lax.fori_loop(0, M, _zero, 0)
    U[M, :] = zrow

    def _scat(jj, carry):
        j0 = jj * LANES
        akv = akbuf[pl.ds(j0, LANES)]
        for l in range(LANES):
            akj = akv[l]
            P[akj, :] = P[akj, :] + tokbuf[j0 + l, :]
        return carry

    lax.fori_loop(0, S // LANES, _scat, 0)

    def _accum(mm, carry):
        m0 = mm * LANES
        memv = membuf[pl.ds(m0, LANES)]
        mqv = mqbuf[pl.ds(m0, LANES)]
        mkv = mkbuf[pl.ds(m0, LANES)]
        urowv = jnp.where(memv >= 0.5, mqv, M)
        for l in range(LANES):
            urow = urowv[l]
            U[urow, :] = U[urow, :] + P[mkv[l], :]
        return carry

    lax.fori_loop(0, M // LANES, _accum, 0)

    def _gath(ii, carry):
        i0 = ii * LANES
        aqv = aqbuf[pl.ds(i0, LANES)]
        for l in range(LANES):
            v = U[aqv[l], :]
            tokbuf[i0 + l, :] = jnp.bitwise_and(
                v.astype(jnp.int32), 1).astype(jnp.float32)
        return carry

    lax.fori_loop(0, S // LANES, _gath, 0)
    pltpu.sync_copy(tokbuf, out_hbm.at[:, pl.ds(c0, LANES)])


_sc_call = functools.partial(
    pl.kernel,
    mesh=plsc.VectorSubcoreMesh(core_axis_name="c", subcore_axis_name="s"),
    out_type=jax.ShapeDtypeStruct((S, IB), jnp.float32),
    compiler_params=pltpu.CompilerParams(use_tc_tiling_on_sc=False),
    scratch_types=[
        pltpu.VMEM((S, LANES), jnp.float32),
        pltpu.VMEM((S,), jnp.int32),
        pltpu.VMEM((S,), jnp.int32),
        pltpu.VMEM((M,), jnp.float32),
        pltpu.VMEM((M,), jnp.int32),
        pltpu.VMEM((M,), jnp.int32),
        pltpu.VMEM((M, LANES), jnp.float32),
        pltpu.VMEM((M + 1, LANES), jnp.float32),
    ],
)(_sc_body)


@jax.jit
def kernel(tokens, memory, connections):
    ib = tokens.shape[1]
    nb = connections.shape[0]
    powers = 2 ** jnp.arange(nb, dtype=jnp.int32)
    qm = connections < ib
    colix = jnp.arange(ib, dtype=jnp.int32)[None, :]
    conn2 = connections[:, None]
    p2 = powers[:, None]
    qm2 = qm[:, None]
    wq = jnp.sum(jnp.where((colix == conn2) & qm2, p2, 0),
                 axis=0).astype(jnp.float32)
    wk = jnp.sum(jnp.where((colix == conn2 - ib) & ~qm2, p2, 0),
                 axis=0).astype(jnp.float32)

    aq, ak = pl.pallas_call(
        _addr_body,
        out_shape=(jax.ShapeDtypeStruct((S,), jnp.int32),
                   jax.ShapeDtypeStruct((S,), jnp.int32)),
    )(tokens, wq, wk)

    qmask = jnp.sum(powers * qm.astype(jnp.int32))
    marange = jnp.arange(M, dtype=jnp.int32)
    mq = jnp.bitwise_and(marange, qmask)
    mk = marange - mq

    return _sc_call(tokens, aq, ak, memory, mq, mk)


# speedup vs baseline: 493.8775x; 1.2952x over previous
import functools

import jax
import jax.numpy as jnp
from jax import lax
from jax.experimental import pallas as pl
from jax.experimental.pallas import tpu as pltpu
from jax.experimental.pallas import tpu_sc as plsc

S = 2048
IB = 512
M = 1024
NC = 2
NS = 16
NT = NC * NS
LANES = 16
W_ROWS = NT + 2


def _addr_body(w_ref, tokens_ref, out_ref):
    acc = lax.dot_general(w_ref[...], tokens_ref[...],
                          (((1,), (1,)), ((), ())),
                          preferred_element_type=jnp.float32)
    out_ref[...] = acc.astype(jnp.int32)


def _sc_body(pk_hbm, mem_hbm, mq_hbm, mk_hbm, out_hbm,
             outbuf, pkv, akv, aqv, membuf, mqbuf, mkbuf, P, U):
    c = lax.axis_index("c")
    s = lax.axis_index("s")
    t = c * NS + s
    c0 = t * LANES

    pltpu.sync_copy(pk_hbm.at[t], pkv)
    pltpu.sync_copy(pk_hbm.at[NT], aqv)
    pltpu.sync_copy(pk_hbm.at[NT + 1], akv)
    pltpu.sync_copy(mem_hbm, membuf)
    pltpu.sync_copy(mq_hbm, mqbuf)
    pltpu.sync_copy(mk_hbm, mkbuf)

    iota = lax.iota(jnp.int32, LANES)
    zero = jnp.zeros((LANES,), jnp.int32)

    def _zero_p(r, carry):
        P[pl.ds(r * LANES, LANES)] = zero
        return carry

    lax.fori_loop(0, M, _zero_p, 0)

    def _zero_u(r, carry):
        U[pl.ds(r * LANES, LANES)] = zero
        return carry

    lax.fori_loop(0, M + 1, _zero_u, 0)

    def _scat(g, carry):
        j0 = g * LANES
        idx = akv[pl.ds(j0, LANES)] * LANES + iota
        tw = pkv[pl.ds(j0, LANES)]
        old = plsc.load_gather(P, [idx])
        plsc.store_scatter(P, [idx], old ^ tw)
        return carry

    lax.fori_loop(0, S // LANES, _scat, 0)

    def _accum(g, carry):
        m0 = g * LANES
        memv = membuf[pl.ds(m0, LANES)]
        urow = jnp.where(memv >= 0.5, mqbuf[pl.ds(m0, LANES)], M)
        pbase = mkbuf[pl.ds(m0, LANES)] * LANES
        uidx = urow * LANES + iota
        pv = plsc.load_gather(P, [pbase])
        for l in range(1, LANES):
            pv = pv ^ plsc.load_gather(P, [pbase + l])
        uv = plsc.load_gather(U, [uidx])
        plsc.store_scatter(U, [uidx], uv ^ pv)
        return carry

    lax.fori_loop(0, M // LANES, _accum, 0)

    def _gath(g, carry):
        j0 = g * LANES
        base = aqv[pl.ds(j0, LANES)] * LANES
        w = plsc.load_gather(U, [base])
        for l in range(1, LANES):
            w = w ^ plsc.load_gather(U, [base + l])
        rows = j0 + iota
        for cc in range(LANES):
            bits = ((w >> cc) & 1).astype(jnp.float32)
            plsc.store_scatter(outbuf, [rows, jnp.full((LANES,), cc,
                                                       jnp.int32)], bits)
        return carry

    lax.fori_loop(0, S // LANES, _gath, 0)
    pltpu.sync_copy(outbuf, out_hbm.at[:, pl.ds(c0, LANES)])


_sc_call = functools.partial(
    pl.kernel,
    mesh=plsc.VectorSubcoreMesh(core_axis_name="c", subcore_axis_name="s"),
    out_type=jax.ShapeDtypeStruct((S, IB), jnp.float32),
    compiler_params=pltpu.CompilerParams(use_tc_tiling_on_sc=False,
                                         needs_layout_passes=False),
    scratch_types=[
        pltpu.VMEM((S, LANES), jnp.float32),
        pltpu.VMEM((S,), jnp.int32),
        pltpu.VMEM((S,), jnp.int32),
        pltpu.VMEM((S,), jnp.int32),
        pltpu.VMEM((M,), jnp.float32),
        pltpu.VMEM((M,), jnp.int32),
        pltpu.VMEM((M,), jnp.int32),
        pltpu.VMEM((M * LANES,), jnp.int32),
        pltpu.VMEM(((M + 1) * LANES,), jnp.int32),
    ],
)(_sc_body)


@jax.jit
def kernel(tokens, memory, connections):
    ib = tokens.shape[1]
    nb = connections.shape[0]
    powers = 2 ** jnp.arange(nb, dtype=jnp.int32)
    qm = connections < ib

    colix = jnp.arange(ib, dtype=jnp.int32)
    tgrp = colix // LANES
    shift_rows = jnp.where(
        tgrp[None, :] == jnp.arange(NT, dtype=jnp.int32)[:, None],
        2 ** (colix % LANES)[None, :], 0)
    conn2 = connections[:, None]
    p2 = powers[:, None]
    qm2 = qm[:, None]
    wq = jnp.sum(jnp.where((colix[None, :] == conn2) & qm2, p2, 0),
                 axis=0)
    wk = jnp.sum(jnp.where((colix[None, :] == conn2 - ib) & ~qm2, p2, 0),
                 axis=0)
    w_all = jnp.concatenate(
        [shift_rows, wq[None, :], wk[None, :]], axis=0).astype(jnp.float32)

    pk = pl.pallas_call(
        _addr_body,
        out_shape=jax.ShapeDtypeStruct((W_ROWS, S), jnp.int32),
    )(w_all, tokens)

    qmask = jnp.sum(powers * qm.astype(jnp.int32))
    marange = jnp.arange(M, dtype=jnp.int32)
    mq = jnp.bitwise_and(marange, qmask)
    mk = marange - mq

    return _sc_call(pk, memory, mq, mk)


# speedup vs baseline: 622.6650x; 1.2608x over previous
import functools

import jax
import jax.numpy as jnp
from jax import lax
from jax.experimental import pallas as pl
from jax.experimental.pallas import tpu as pltpu
from jax.experimental.pallas import tpu_sc as plsc

S = 2048
IB = 512
NB = 10
M = 1024
NC = 2
NS = 16
NT = NC * NS
LANES = 16
W_ROWS = NT + 2


def _addr_body(tokens_ref, conn_ref, pk_ref, mqk_ref):
    conn2 = conn_ref[...]
    powers = 1 << lax.broadcasted_iota(jnp.int32, (NB, 1), 0)
    qm = conn2 < IB
    colix = lax.broadcasted_iota(jnp.int32, (NB, IB), 1)
    wq = jnp.sum(jnp.where((colix == conn2) & qm, powers, 0),
                 axis=0, keepdims=True)
    wk = jnp.sum(jnp.where((colix == conn2 - IB) & ~qm, powers, 0),
                 axis=0, keepdims=True)

    tcol = lax.broadcasted_iota(jnp.int32, (NT, IB), 1)
    trow = lax.broadcasted_iota(jnp.int32, (NT, IB), 0)
    shift_mat = jnp.where(tcol // LANES == trow, 1 << (tcol % LANES), 0)
    w_all = jnp.concatenate([shift_mat, wq, wk], axis=0).astype(jnp.float32)

    pk_ref[...] = lax.dot_general(
        w_all, tokens_ref[...], (((1,), (1,)), ((), ())),
        preferred_element_type=jnp.float32).astype(jnp.int32)

    qmask = jnp.sum(jnp.where(qm, powers, 0))
    mar = lax.broadcasted_iota(jnp.int32, (1, M), 1)
    mq = jnp.bitwise_and(mar, qmask)
    mqk_ref[pl.ds(0, 1), :] = mq
    mqk_ref[pl.ds(1, 1), :] = mar - mq


def _sc_body(pk_hbm, mem_hbm, mqk_hbm, out_hbm,
             outbuf, pkv, aqv, akv, membuf, mqbuf, mkbuf, P, U, sem):
    c = lax.axis_index("c")
    s = lax.axis_index("s")
    t = c * NS + s
    c0 = t * LANES

    copies = [
        pltpu.async_copy(pk_hbm.at[t], pkv, sem),
        pltpu.async_copy(pk_hbm.at[NT], aqv, sem),
        pltpu.async_copy(pk_hbm.at[NT + 1], akv, sem),
        pltpu.async_copy(mem_hbm, membuf, sem),
        pltpu.async_copy(mqk_hbm.at[0], mqbuf, sem),
        pltpu.async_copy(mqk_hbm.at[1], mkbuf, sem),
    ]

    iota = lax.iota(jnp.int32, LANES)
    zero = jnp.zeros((LANES,), jnp.int32)

    @plsc.parallel_loop(0, (M * LANES) // 128, unroll=2)
    def _zero_p(r):
        for k in range(8):
            P[pl.ds(r * 128 + k * LANES, LANES)] = zero

    @plsc.parallel_loop(0, (M * LANES) // 128, unroll=2)
    def _zero_u(r):
        for k in range(8):
            U[pl.ds(r * 128 + k * LANES, LANES)] = zero

    U[pl.ds(M * LANES, LANES)] = zero

    for cp in copies:
        cp.wait()

    def _scat(g, carry):
        j0 = g * LANES
        idx = akv[pl.ds(j0, LANES)] * LANES + iota
        tw = pkv[pl.ds(j0, LANES)]
        old = plsc.load_gather(P, [idx])
        plsc.store_scatter(P, [idx], old ^ tw)
        return carry

    lax.fori_loop(0, S // LANES, _scat, 0)

    def _accum(g, carry):
        m0 = g * LANES
        memv = membuf[pl.ds(m0, LANES)]
        urow = jnp.where(memv >= 0.5, mqbuf[pl.ds(m0, LANES)], M)
        pbase = mkbuf[pl.ds(m0, LANES)] * LANES
        uidx = urow * LANES + iota
        pv = plsc.load_gather(P, [pbase])
        for l in range(1, LANES):
            pv = pv ^ plsc.load_gather(P, [pbase + l])
        uv = plsc.load_gather(U, [uidx])
        plsc.store_scatter(U, [uidx], uv ^ pv)
        return carry

    lax.fori_loop(0, M // LANES, _accum, 0)

    @plsc.parallel_loop(0, S // LANES, unroll=2)
    def _gath(g):
        j0 = g * LANES
        base = aqv[pl.ds(j0, LANES)] * LANES
        w = plsc.load_gather(U, [base])
        for l in range(1, LANES):
            w = w ^ plsc.load_gather(U, [base + l])
        rows = j0 + iota
        for cc in range(LANES):
            bits = ((w >> cc) & 1).astype(jnp.float32)
            plsc.store_scatter(outbuf, [rows, jnp.full((LANES,), cc,
                                                       jnp.int32)], bits)

    pltpu.sync_copy(outbuf, out_hbm.at[:, pl.ds(c0, LANES)])


_sc_call = functools.partial(
    pl.kernel,
    mesh=plsc.VectorSubcoreMesh(core_axis_name="c", subcore_axis_name="s"),
    out_type=jax.ShapeDtypeStruct((S, IB), jnp.float32),
    compiler_params=pltpu.CompilerParams(use_tc_tiling_on_sc=False,
                                         needs_layout_passes=False),
    scratch_types=[
        pltpu.VMEM((S, LANES), jnp.float32),
        pltpu.VMEM((S,), jnp.int32),
        pltpu.VMEM((S,), jnp.int32),
        pltpu.VMEM((S,), jnp.int32),
        pltpu.VMEM((M,), jnp.float32),
        pltpu.VMEM((M,), jnp.int32),
        pltpu.VMEM((M,), jnp.int32),
        pltpu.VMEM((M * LANES,), jnp.int32),
        pltpu.VMEM(((M + 1) * LANES,), jnp.int32),
        pltpu.SemaphoreType.DMA,
    ],
)(_sc_body)


@jax.jit
def kernel(tokens, memory, connections):
    pk, mqk = pl.pallas_call(
        _addr_body,
        out_shape=(jax.ShapeDtypeStruct((W_ROWS, S), jnp.int32),
                   jax.ShapeDtypeStruct((2, M), jnp.int32)),
    )(tokens, connections.reshape(NB, 1))
    return _sc_call(pk, memory, mqk)


# speedup vs baseline: 800.4269x; 1.2855x over previous
import functools

import jax
import jax.numpy as jnp
from jax import lax
from jax.experimental import pallas as pl
from jax.experimental.pallas import tpu as pltpu
from jax.experimental.pallas import tpu_sc as plsc

S = 2048
IB = 512
NB = 10
M = 1024
NC = 2
NS = 16
NT = NC * NS
LANES = 16
W_ROWS = NT + 2


def _addr_body(tokens_ref, conn_ref, pk_ref, mqk_ref):
    conn2 = conn_ref[...]
    powers = 1 << lax.broadcasted_iota(jnp.int32, (NB, 1), 0)
    qm = conn2 < IB
    colix = lax.broadcasted_iota(jnp.int32, (NB, IB), 1)
    wq = jnp.sum(jnp.where((colix == conn2) & qm, powers, 0),
                 axis=0, keepdims=True)
    wk = jnp.sum(jnp.where((colix == conn2 - IB) & ~qm, powers, 0),
                 axis=0, keepdims=True)

    tcol = lax.broadcasted_iota(jnp.int32, (NT, IB), 1)
    trow = lax.broadcasted_iota(jnp.int32, (NT, IB), 0)
    shift_mat = jnp.where(tcol // LANES == trow, 1 << (tcol % LANES), 0)
    w_all = jnp.concatenate([shift_mat, wq, wk], axis=0).astype(jnp.float32)

    pk_ref[...] = lax.dot_general(
        w_all, tokens_ref[...], (((1,), (1,)), ((), ())),
        preferred_element_type=jnp.float32).astype(jnp.int32)

    qmask = jnp.sum(jnp.where(qm, powers, 0))
    mar = lax.broadcasted_iota(jnp.int32, (1, M), 1)
    mq = jnp.bitwise_and(mar, qmask)
    mqk_ref[pl.ds(0, 1), :] = mq
    mqk_ref[pl.ds(1, 1), :] = mar - mq


def _sc_body(pk_hbm, mem_hbm, mqk_hbm, out_hbm,
             outbuf, pkv, aqv, akv, membuf, mqbuf, mkbuf, P, U, sem):
    c = lax.axis_index("c")
    s = lax.axis_index("s")
    t = c * NS + s
    c0 = t * LANES

    copies = [
        pltpu.async_copy(pk_hbm.at[t], pkv, sem),
        pltpu.async_copy(pk_hbm.at[NT], aqv, sem),
        pltpu.async_copy(pk_hbm.at[NT + 1], akv, sem),
        pltpu.async_copy(mem_hbm, membuf, sem),
        pltpu.async_copy(mqk_hbm.at[0], mqbuf, sem),
        pltpu.async_copy(mqk_hbm.at[1], mkbuf, sem),
    ]

    iota = lax.iota(jnp.int32, LANES)
    zero = jnp.zeros((LANES,), jnp.int32)
    rot = [(l + iota) & (LANES - 1) for l in range(LANES)]

    @plsc.parallel_loop(0, (M * LANES) // 128, unroll=2)
    def _zero_p(r):
        for k in range(8):
            P[pl.ds(r * 128 + k * LANES, LANES)] = zero

    @plsc.parallel_loop(0, (M * LANES) // 128, unroll=2)
    def _zero_u(r):
        for k in range(8):
            U[pl.ds(r * 128 + k * LANES, LANES)] = zero

    U[pl.ds(M * LANES, LANES)] = zero

    for cp in copies:
        cp.wait()

    def _scat(g, carry):
        j0 = g * LANES
        idx = akv[pl.ds(j0, LANES)] * LANES + iota
        tw = pkv[pl.ds(j0, LANES)]
        old = plsc.load_gather(P, [idx])
        plsc.store_scatter(P, [idx], old ^ tw)
        return carry

    lax.fori_loop(0, S // LANES, _scat, 0)

    def _accum(g, carry):
        m0 = g * LANES
        memv = membuf[pl.ds(m0, LANES)]
        urow = jnp.where(memv >= 0.5, mqbuf[pl.ds(m0, LANES)], M)
        pbase = mkbuf[pl.ds(m0, LANES)] * LANES
        uidx = urow * LANES + iota
        pv = plsc.load_gather(P, [pbase + rot[0]])
        for l in range(1, LANES):
            pv = pv ^ plsc.load_gather(P, [pbase + rot[l]])
        uv = plsc.load_gather(U, [uidx])
        plsc.store_scatter(U, [uidx], uv ^ pv)
        return carry

    lax.fori_loop(0, M // LANES, _accum, 0)

    @plsc.parallel_loop(0, S // LANES, unroll=2)
    def _gath(g):
        j0 = g * LANES
        base = aqv[pl.ds(j0, LANES)] * LANES
        w = plsc.load_gather(U, [base + rot[0]])
        for l in range(1, LANES):
            w = w ^ plsc.load_gather(U, [base + rot[l]])
        rows = j0 + iota
        for cc in range(LANES):
            bits = ((w >> rot[cc]) & 1).astype(jnp.float32)
            plsc.store_scatter(outbuf, [rows, rot[cc]], bits)

    pltpu.sync_copy(outbuf, out_hbm.at[:, pl.ds(c0, LANES)])


_sc_call = functools.partial(
    pl.kernel,
    mesh=plsc.VectorSubcoreMesh(core_axis_name="c", subcore_axis_name="s"),
    out_type=jax.ShapeDtypeStruct((S, IB), jnp.float32),
    compiler_params=pltpu.CompilerParams(use_tc_tiling_on_sc=False,
                                         needs_layout_passes=False),
    scratch_types=[
        pltpu.VMEM((S, LANES), jnp.float32),
        pltpu.VMEM((S,), jnp.int32),
        pltpu.VMEM((S,), jnp.int32),
        pltpu.VMEM((S,), jnp.int32),
        pltpu.VMEM((M,), jnp.float32),
        pltpu.VMEM((M,), jnp.int32),
        pltpu.VMEM((M,), jnp.int32),
        pltpu.VMEM((M * LANES,), jnp.int32),
        pltpu.VMEM(((M + 1) * LANES,), jnp.int32),
        pltpu.SemaphoreType.DMA,
    ],
)(_sc_body)


@jax.jit
def kernel(tokens, memory, connections):
    pk, mqk = pl.pallas_call(
        _addr_body,
        out_shape=(jax.ShapeDtypeStruct((W_ROWS, S), jnp.int32),
                   jax.ShapeDtypeStruct((2, M), jnp.int32)),
    )(tokens, connections.reshape(NB, 1))
    return _sc_call(pk, memory, mqk)


# speedup vs baseline: 855.0381x; 1.0682x over previous
import functools

import jax
import jax.numpy as jnp
from jax import lax
from jax.experimental import pallas as pl
from jax.experimental.pallas import tpu as pltpu
from jax.experimental.pallas import tpu_sc as plsc

S = 2048
IB = 512
NB = 10
M = 1024
NC = 2
NS = 16
NT = NC * NS
LANES = 16
W_ROWS = NT + 2


def _addr_body(tokens_ref, conn_ref, pk_ref, mqk_ref):
    conn2 = conn_ref[...]
    powers = 1 << lax.broadcasted_iota(jnp.int32, (NB, 1), 0)
    qm = conn2 < IB
    colix = lax.broadcasted_iota(jnp.int32, (NB, IB), 1)
    wq = jnp.sum(jnp.where((colix == conn2) & qm, powers, 0),
                 axis=0, keepdims=True)
    wk = jnp.sum(jnp.where((colix == conn2 - IB) & ~qm, powers, 0),
                 axis=0, keepdims=True)

    tcol = lax.broadcasted_iota(jnp.int32, (NT, IB), 1)
    trow = lax.broadcasted_iota(jnp.int32, (NT, IB), 0)
    shift_mat = jnp.where(tcol % NT == trow, 1 << (tcol // NT), 0)
    w_all = jnp.concatenate([shift_mat, wq, wk], axis=0).astype(jnp.float32)

    pk_ref[...] = lax.dot_general(
        w_all, tokens_ref[...], (((1,), (1,)), ((), ())),
        preferred_element_type=jnp.float32).astype(jnp.int32)

    qmask = jnp.sum(jnp.where(qm, powers, 0))
    mar = lax.broadcasted_iota(jnp.int32, (1, M), 1)
    mq = jnp.bitwise_and(mar, qmask)
    mqk_ref[pl.ds(0, 1), :] = mq
    mqk_ref[pl.ds(1, 1), :] = mar - mq


def _unpack_body(w_ref, out_ref):
    xt = jnp.transpose(w_ref[...], (1, 0))
    parts = [((xt >> b) & 1).astype(jnp.float32) for b in range(LANES)]
    out_ref[...] = jnp.concatenate(parts, axis=1)


def _sc_body(pk_hbm, mem_hbm, mqk_hbm, out_hbm,
             outw, pkv, aqv, akv, membuf, mqbuf, mkbuf, P, U, sem):
    c = lax.axis_index("c")
    s = lax.axis_index("s")
    t = c * NS + s

    copies = [
        pltpu.async_copy(pk_hbm.at[t], pkv, sem),
        pltpu.async_copy(pk_hbm.at[NT], aqv, sem),
        pltpu.async_copy(pk_hbm.at[NT + 1], akv, sem),
        pltpu.async_copy(mem_hbm, membuf, sem),
        pltpu.async_copy(mqk_hbm.at[0], mqbuf, sem),
        pltpu.async_copy(mqk_hbm.at[1], mkbuf, sem),
    ]

    iota = lax.iota(jnp.int32, LANES)
    zero = jnp.zeros((LANES,), jnp.int32)
    rot = [(l + iota) & (LANES - 1) for l in range(LANES)]

    @plsc.parallel_loop(0, (M * LANES) // 128, unroll=2)
    def _zero_p(r):
        for k in range(8):
            P[pl.ds(r * 128 + k * LANES, LANES)] = zero

    @plsc.parallel_loop(0, (M * LANES) // 128, unroll=2)
    def _zero_u(r):
        for k in range(8):
            U[pl.ds(r * 128 + k * LANES, LANES)] = zero

    U[pl.ds(M * LANES, LANES)] = zero

    for cp in copies:
        cp.wait()

    def _scat(g, carry):
        j0 = g * LANES
        idx = akv[pl.ds(j0, LANES)] * LANES + iota
        tw = pkv[pl.ds(j0, LANES)]
        old = plsc.load_gather(P, [idx])
        plsc.store_scatter(P, [idx], old ^ tw)
        return carry

    lax.fori_loop(0, S // LANES, _scat, 0)

    def _accum(g, carry):
        m0 = g * LANES
        memv = membuf[pl.ds(m0, LANES)]
        urow = jnp.where(memv >= 0.5, mqbuf[pl.ds(m0, LANES)], M)
        pbase = mkbuf[pl.ds(m0, LANES)] * LANES
        uidx = urow * LANES + iota
        pv = plsc.load_gather(P, [pbase + rot[0]])
        for l in range(1, LANES):
            pv = pv ^ plsc.load_gather(P, [pbase + rot[l]])
        uv = plsc.load_gather(U, [uidx])
        plsc.store_scatter(U, [uidx], uv ^ pv)
        return carry

    lax.fori_loop(0, M // LANES, _accum, 0)

    @plsc.parallel_loop(0, S // LANES, unroll=2)
    def _gath(g):
        j0 = g * LANES
        base = aqv[pl.ds(j0, LANES)] * LANES
        w = plsc.load_gather(U, [base + rot[0]])
        for l in range(1, LANES):
            w = w ^ plsc.load_gather(U, [base + rot[l]])
        outw[pl.ds(j0, LANES)] = w

    pltpu.sync_copy(outw, out_hbm.at[t])


_sc_call = functools.partial(
    pl.kernel,
    mesh=plsc.VectorSubcoreMesh(core_axis_name="c", subcore_axis_name="s"),
    out_type=jax.ShapeDtypeStruct((NT, S), jnp.int32),
    compiler_params=pltpu.CompilerParams(use_tc_tiling_on_sc=False,
                                         needs_layout_passes=False),
    scratch_types=[
        pltpu.VMEM((S,), jnp.int32),
        pltpu.VMEM((S,), jnp.int32),
        pltpu.VMEM((S,), jnp.int32),
        pltpu.VMEM((S,), jnp.int32),
        pltpu.VMEM((M,), jnp.float32),
        pltpu.VMEM((M,), jnp.int32),
        pltpu.VMEM((M,), jnp.int32),
        pltpu.VMEM((M * LANES,), jnp.int32),
        pltpu.VMEM(((M + 1) * LANES,), jnp.int32),
        pltpu.SemaphoreType.DMA,
    ],
)(_sc_body)


@jax.jit
def kernel(tokens, memory, connections):
    pk, mqk = pl.pallas_call(
        _addr_body,
        out_shape=(jax.ShapeDtypeStruct((W_ROWS, S), jnp.int32),
                   jax.ShapeDtypeStruct((2, M), jnp.int32)),
    )(tokens, connections.reshape(NB, 1))
    w_out = _sc_call(pk, memory, mqk)
    return pl.pallas_call(
        _unpack_body,
        out_shape=jax.ShapeDtypeStruct((S, IB), jnp.float32),
    )(w_out)
